# baseline (speedup 1.0000x reference)
import functools

import jax
import jax.numpy as jnp
from jax import lax
from jax.experimental import pallas as pl
from jax.experimental.pallas import tpu as pltpu
from jax.experimental.pallas import tpu_sc as plsc

VOCAB = 100000
EMB = 64
B = 4096
L = 50

NW = 32
BTOT = B * L
B_PER_W = BTOT // NW
CW = 128
NCH = B_PER_W // CW
NBUF = 14
G = 7


def _emb_body(idx_hbm, tab_hbm, out_hbm, idx_v, rows_v, gsem, ssem):
    wid = lax.axis_index("s") * 2 + lax.axis_index("c")
    pltpu.sync_copy(idx_hbm.at[wid], idx_v)
    base = wid * B_PER_W

    def fire_gather(i, slot):
        pltpu.async_copy(tab_hbm.at[idx_v.at[i]], rows_v.at[slot], gsem)

    def wait_gather(slot):
        pltpu.make_async_copy(tab_hbm.at[pl.ds(0, CW)], rows_v.at[slot], gsem).wait()

    def fire_scatter(i, slot):
        pltpu.async_copy(rows_v.at[slot], out_hbm.at[pl.ds(base + i * CW, CW)], ssem)

    def drain_scatter():
        pltpu.make_async_copy(out_hbm.at[pl.ds(0, CW)], rows_v.at[0], ssem).wait()

    for b in range(G):
        fire_gather(b, b)

    for b in range(G):
        wait_gather(b)
        fire_scatter(b, b)
        fire_gather(b + G, b + G)

    def steady(i, b):
        drain_scatter()
        fire_gather(i + G, b)
        wait_gather((b + G) % NBUF)
        fire_scatter(i, (b + G) % NBUF)

    ngroups = (NCH - 2 * G) // NBUF

    def group(g, carry):
        i0 = G + g * NBUF
        for b in range(NBUF):
            steady(i0 + b, b)
        return carry

    lax.fori_loop(0, ngroups, group, 0)
    for r in range((NCH - 2 * G) % NBUF):
        steady(G + ngroups * NBUF + r, r)

    for b in range(G):
        i = NCH - G + b
        wait_gather(i % NBUF)
        fire_scatter(i, i % NBUF)

    for _ in range(2 * G):
        drain_scatter()


TB = 800
NTB = VOCAB // TB


def _tr_body(wt_hbm, tab_hbm, in_v, out_v, sem):
    wid = lax.axis_index("s") * 2 + lax.axis_index("c")
    lanes = lax.iota(jnp.int32, 16)

    def do_block(blk):
        r0 = blk * TB
        pltpu.sync_copy(wt_hbm.at[:, pl.ds(r0, TB)], in_v)

        for c in range(EMB):
            col = jnp.full((16,), c, jnp.int32)

            def r_loop(r, rows):
                vals = plsc.load_gather(in_v, [col, rows])
                plsc.store_scatter(out_v, [rows, col], vals)
                return rows + 16

            lax.fori_loop(0, TB // 16, r_loop, lanes)
        pltpu.sync_copy(out_v, tab_hbm.at[pl.ds(r0, TB)])

    for j in range(5):
        blk = wid + NW * j

        @pl.when(blk < NTB)
        def _():
            do_block(blk)


@jax.jit
def _transpose(wt):
    mesh = plsc.VectorSubcoreMesh(core_axis_name="c", subcore_axis_name="s")
    f = pl.kernel(
        _tr_body,
        out_type=jax.ShapeDtypeStruct((VOCAB, EMB), jnp.float32),
        mesh=mesh,
        scratch_types=[
            pltpu.VMEM((EMB, TB), jnp.float32),
            pltpu.VMEM((TB, EMB), jnp.float32),
            pltpu.SemaphoreType.DMA,
        ],
        compiler_params=pltpu.CompilerParams(
            use_tc_tiling_on_sc=False, needs_layout_passes=False),
    )
    return f(wt)


@jax.jit
def _emb(xw, wemb):
    mesh = plsc.VectorSubcoreMesh(core_axis_name="c", subcore_axis_name="s")
    f = pl.kernel(
        _emb_body,
        out_type=jax.ShapeDtypeStruct((BTOT, EMB), jnp.float32),
        mesh=mesh,
        scratch_types=[
            pltpu.VMEM((NCH, CW), jnp.int32),
            pltpu.VMEM((NBUF, CW, EMB), jnp.float32),
            pltpu.SemaphoreType.DMA,
            pltpu.SemaphoreType.DMA,
        ],
        compiler_params=pltpu.CompilerParams(use_tc_tiling_on_sc=False),
    )
    return f(xw, wemb)


def kernel(x, wemb):
    xw = x.T.reshape(NW, NCH, CW).astype(jnp.int32)
    tab = _transpose(wemb.T)
    out = _emb(xw, tab)
    return out.reshape(L, B, EMB).transpose(1, 0, 2)
